# baseline (speedup 1.0000x reference)
import functools

import jax
import jax.numpy as jnp
from jax import lax
from jax.experimental import pallas as pl
from jax.experimental.pallas import tpu as pltpu
from jax.experimental.pallas import tpu_sc as plsc

_RC = 5.0
_B, _N, _D = 50, 2000, 3
_E = 6400000
_CHS = 11
_CH = 1 << _CHS
_NPAD = 2048
_L = 16
_NC, _NS = 2, 16
_NW = _NC * _NS
_NG = _CH // _L
_ROWS = _NPAD // _L
_SEGPAD = 64
_NCHUNK = _E // _CH
_CPW = -(-_NCHUNK // _NW)


def _lookup(segv, i):
    vec = segv[pl.ds((i >> 4) * _L, _L)]
    onehot = lax.iota(jnp.int32, _L) == jnp.full((_L,), i & (_L - 1), jnp.int32)
    return jnp.max(jnp.where(onehot, vec, 0))


def _last_idx(segv, val, strict):
    best = jnp.int32(0)
    for r in range(_SEGPAD // _L):
        vec = segv[pl.ds(r * _L, _L)]
        cond = (vec < val) if strict else (vec <= val)
        idx = lax.iota(jnp.int32, _L) + (r * _L + 1)
        best = jnp.maximum(best, jnp.max(jnp.where(cond, idx, 0)))
    return best - 1


def _force_body(pos_hbm, seg_hbm, src_hbm, dst_hbm, shx_hbm, shy_hbm,
                shz_hbm, zeros_hbm, out_hbm, segv, posv, accv,
                srcv0, dstv0, shxv0, shyv0, shzv0,
                srcv1, dstv1, shxv1, shyv1, shzv1, sem0, sem1):
    cid = lax.axis_index("c")
    sid = lax.axis_index("s")
    wid = sid * _NC + cid

    pltpu.sync_copy(seg_hbm, segv)
    iota16 = lax.iota(jnp.int32, _L)
    sems = (sem0, sem1)
    hbms = (src_hbm, dst_hbm, shx_hbm, shy_hbm, shz_hbm)
    bufs = ((srcv0, dstv0, shxv0, shyv0, shzv0),
            (srcv1, dstv1, shxv1, shyv1, shzv1))

    def fire(kk, p):
        for h, v in zip(hbms, bufs[p]):
            pltpu.async_copy(h.at[pl.ds(kk * _CH, _CH)], v, sems[p])

    def drain(p):
        for h, v in zip(hbms, bufs[p]):
            pltpu.make_async_copy(h.at[pl.ds(0, _CH)], v, sems[p]).wait()

    ck0 = wid * _CPW
    ck1 = jnp.minimum(ck0 + _CPW, _NCHUNK)
    elo = ck0 * _CH
    ehi = ck1 * _CH
    fb = _last_idx(segv, elo, False)
    lb = _last_idx(segv, ehi, True)

    def seg_body(t, carry):
        b = fb + t
        blo = _lookup(segv, b)
        bhi = _lookup(segv, b + 1)
        clo = jnp.maximum(blo, elo)
        chi = jnp.minimum(bhi, ehi)

        @pl.when(chi > clo)
        def _():
            k0 = clo >> _CHS
            k1 = (chi + (_CH - 1)) >> _CHS
            fire(k0, 0)
            pltpu.sync_copy(zeros_hbm, accv)
            pltpu.sync_copy(pos_hbm.at[b], posv)

            def compute_chunk(kk, p):
                base = kk * _CH
                srcb, dstb, shxb, shyb, shzb = bufs[p]

                def group_body(j, carry3):
                    off = j * _L
                    s16 = srcb[pl.ds(off, _L)]
                    d16 = dstb[pl.ds(off, _L)]
                    eabs = iota16 + (base + off)
                    valid = (eabs >= clo) & (eabs < chi)
                    s_hi = s16 >> 4
                    s_lo = s16 & (_L - 1)
                    d_hi = d16 >> 4
                    d_lo = d16 & (_L - 1)
                    disp = []
                    for c, shv in ((0, shxb), (1, shyb), (2, shzb)):
                        crow = c * _ROWS
                        ps = plsc.load_gather(posv, [s_hi + crow, s_lo])
                        pd = plsc.load_gather(posv, [d_hi + crow, d_lo])
                        disp.append(pd - ps + shv[pl.ds(off, _L)])
                    dx, dy, dz = disp
                    s = dx * dx + dy * dy + dz * dz
                    ib = lax.bitcast_convert_type(s, jnp.int32)
                    y = lax.bitcast_convert_type(
                        jnp.int32(0x5F3759DF) - (ib >> 1), jnp.float32)
                    y = y * (1.5 - 0.5 * s * y * y)
                    y = y * (1.5 - 0.5 * s * y * y)
                    pref = 2.0 - (2.0 * _RC) * y
                    for c in range(3):
                        plsc.addupdate_scatter(
                            accv, [s_hi + c * _ROWS, s_lo],
                            pref * disp[c], mask=valid)
                    return carry3

                lax.fori_loop(0, _NG, group_body, 0)

            def pair_body(i, carry2):
                for p in range(2):
                    kk = k0 + 2 * i + p

                    @pl.when(kk < k1)
                    def _():
                        @pl.when(kk + 1 < k1)
                        def _():
                            fire(kk + 1, 1 - p)

                        drain(p)
                        compute_chunk(kk, p)
                return carry2

            lax.fori_loop(0, (k1 - k0 + 1) >> 1, pair_body, 0)
            pltpu.sync_copy(accv, out_hbm.at[wid, b])

        return carry

    lax.fori_loop(0, lb - fb + 1, seg_body, 0)


_sc_forces = functools.partial(
    pl.kernel,
    out_type=jax.ShapeDtypeStruct((_NW, _B, 3 * _ROWS, _L), jnp.float32),
    mesh=plsc.VectorSubcoreMesh(core_axis_name="c", subcore_axis_name="s",
                                num_cores=_NC, num_subcores=_NS),
    scratch_types=[
        pltpu.VMEM((_SEGPAD,), jnp.int32),
        pltpu.VMEM((3 * _ROWS, _L), jnp.float32),
        pltpu.VMEM((3 * _ROWS, _L), jnp.float32),
        pltpu.VMEM((_CH,), jnp.int32),
        pltpu.VMEM((_CH,), jnp.int32),
        pltpu.VMEM((_CH,), jnp.float32),
        pltpu.VMEM((_CH,), jnp.float32),
        pltpu.VMEM((_CH,), jnp.float32),
        pltpu.VMEM((_CH,), jnp.int32),
        pltpu.VMEM((_CH,), jnp.int32),
        pltpu.VMEM((_CH,), jnp.float32),
        pltpu.VMEM((_CH,), jnp.float32),
        pltpu.VMEM((_CH,), jnp.float32),
        pltpu.SemaphoreType.DMA,
        pltpu.SemaphoreType.DMA,
    ],
    compiler_params=pltpu.CompilerParams(needs_layout_passes=False),
)(_force_body)


def kernel(relative_coordinates, basis_vectors, raw_scores_X, shifts,
           edge_src, edge_dst, edge_batch):
    b, n, d = relative_coordinates.shape
    pos = jnp.einsum('bnd,bde->bne', relative_coordinates, basis_vectors)
    pos_t = jnp.swapaxes(pos, 1, 2)
    pos_t = jnp.pad(pos_t, ((0, 0), (0, 0), (0, _NPAD - n)))
    pos_pad = pos_t.reshape(b, 3 * _ROWS, _L)
    seg = jnp.searchsorted(
        edge_batch, jnp.arange(b + 1, dtype=edge_batch.dtype)).astype(jnp.int32)
    seg_pad = jnp.concatenate(
        [seg, jnp.full((_SEGPAD - (b + 1),), _E, jnp.int32)])
    zeros = jnp.zeros((3 * _ROWS, _L), jnp.float32)
    shifts_t = shifts.T
    planes = _sc_forces(pos_pad, seg_pad,
                        edge_src.astype(jnp.int32),
                        edge_dst.astype(jnp.int32),
                        shifts_t[0], shifts_t[1], shifts_t[2], zeros)
    ck0 = jnp.minimum(jnp.arange(_NW) * _CPW, _NCHUNK)
    elo = ck0 * _CH
    ehi = jnp.minimum(ck0 + _CPW, _NCHUNK) * _CH
    mask = (jnp.minimum(seg[None, 1:], ehi[:, None])
            > jnp.maximum(seg[None, :-1], elo[:, None]))
    acc = jnp.where(mask[:, :, None, None], planes, 0.0).sum(axis=0)
    forces = jnp.transpose(acc.reshape(b, 3, _NPAD)[:, :, :n], (0, 2, 1))
    rel_forces = jnp.einsum(
        'bnd,bde->bne', forces, jnp.linalg.inv(basis_vectors))
    return raw_scores_X + rel_forces


# speedup vs baseline: 1.0818x; 1.0818x over previous
import functools

import jax
import jax.numpy as jnp
from jax import lax
from jax.experimental import pallas as pl
from jax.experimental.pallas import tpu as pltpu
from jax.experimental.pallas import tpu_sc as plsc

_RC = 5.0
_B, _N, _D = 50, 2000, 3
_E = 6400000
_CHS = 10
_CH = 1 << _CHS
_NPAD = 2048
_L = 16
_NC, _NS = 2, 16
_NW = _NC * _NS
_NG = _CH // _L
_ROWS = _NPAD // _L
_SEGPAD = 64
_NCHUNK = _E // _CH
_CPW = -(-_NCHUNK // _NW)


def _lookup(segv, i):
    vec = segv[pl.ds((i >> 4) * _L, _L)]
    onehot = lax.iota(jnp.int32, _L) == jnp.full((_L,), i & (_L - 1), jnp.int32)
    return jnp.max(jnp.where(onehot, vec, 0))


def _last_idx(segv, val, strict):
    best = jnp.int32(0)
    for r in range(_SEGPAD // _L):
        vec = segv[pl.ds(r * _L, _L)]
        cond = (vec < val) if strict else (vec <= val)
        idx = lax.iota(jnp.int32, _L) + (r * _L + 1)
        best = jnp.maximum(best, jnp.max(jnp.where(cond, idx, 0)))
    return best - 1


def _force_body(pos_hbm, seg_hbm, src_hbm, dst_hbm, shx_hbm, shy_hbm,
                shz_hbm, zeros_hbm, out_hbm, segv, posv, accv,
                srcv0, dstv0, shxv0, shyv0, shzv0,
                srcv1, dstv1, shxv1, shyv1, shzv1, sem0, sem1):
    cid = lax.axis_index("c")
    sid = lax.axis_index("s")
    wid = sid * _NC + cid

    pltpu.sync_copy(seg_hbm, segv)
    iota16 = lax.iota(jnp.int32, _L)
    sems = (sem0, sem1)
    hbms = (src_hbm, dst_hbm, shx_hbm, shy_hbm, shz_hbm)
    bufs = ((srcv0, dstv0, shxv0, shyv0, shzv0),
            (srcv1, dstv1, shxv1, shyv1, shzv1))

    def fire(kk, p):
        for h, v in zip(hbms, bufs[p]):
            pltpu.async_copy(h.at[pl.ds(kk * _CH, _CH)], v, sems[p])

    def drain(p):
        for h, v in zip(hbms, bufs[p]):
            pltpu.make_async_copy(h.at[pl.ds(0, _CH)], v, sems[p]).wait()

    ck0 = wid * _CPW
    ck1 = jnp.minimum(ck0 + _CPW, _NCHUNK)
    elo = ck0 * _CH
    ehi = ck1 * _CH
    fb = _last_idx(segv, elo, False)
    lb = _last_idx(segv, ehi, True)

    def seg_body(t, carry):
        b = fb + t
        blo = _lookup(segv, b)
        bhi = _lookup(segv, b + 1)
        clo = jnp.maximum(blo, elo)
        chi = jnp.minimum(bhi, ehi)

        @pl.when(chi > clo)
        def _():
            k0 = clo >> _CHS
            k1 = (chi + (_CH - 1)) >> _CHS
            fire(k0, 0)
            pltpu.sync_copy(zeros_hbm, accv)
            pltpu.sync_copy(pos_hbm.at[b], posv)

            def compute_chunk(kk, p):
                base = kk * _CH
                srcb, dstb, shxb, shyb, shzb = bufs[p]

                def group_body(j, carry3):
                    off = j * _L
                    s16 = srcb[pl.ds(off, _L)]
                    d16 = dstb[pl.ds(off, _L)]
                    eabs = iota16 + (base + off)
                    valid = (eabs >= clo) & (eabs < chi)
                    s_hi = s16 >> 4
                    s_lo = s16 & (_L - 1)
                    d_hi = d16 >> 4
                    d_lo = d16 & (_L - 1)
                    disp = []
                    for c, shv in ((0, shxb), (1, shyb), (2, shzb)):
                        crow = c * _ROWS
                        ps = plsc.load_gather(posv, [s_hi + crow, s_lo])
                        pd = plsc.load_gather(posv, [d_hi + crow, d_lo])
                        disp.append(pd - ps + shv[pl.ds(off, _L)])
                    dx, dy, dz = disp
                    s = dx * dx + dy * dy + dz * dz
                    ib = lax.bitcast_convert_type(s, jnp.int32)
                    y = lax.bitcast_convert_type(
                        jnp.int32(0x5F3759DF) - (ib >> 1), jnp.float32)
                    y = y * (1.5 - 0.5 * s * y * y)
                    pref = 2.0 - (2.0 * _RC) * y
                    for c in range(3):
                        plsc.addupdate_scatter(
                            accv, [s_hi + c * _ROWS, s_lo],
                            pref * disp[c], mask=valid)
                    return carry3

                lax.fori_loop(0, _NG, group_body, 0)

            def pair_body(i, carry2):
                for p in range(2):
                    kk = k0 + 2 * i + p

                    @pl.when(kk < k1)
                    def _():
                        @pl.when(kk + 1 < k1)
                        def _():
                            fire(kk + 1, 1 - p)

                        drain(p)
                        compute_chunk(kk, p)
                return carry2

            lax.fori_loop(0, (k1 - k0 + 1) >> 1, pair_body, 0)
            pltpu.sync_copy(accv, out_hbm.at[wid, b])

        return carry

    lax.fori_loop(0, lb - fb + 1, seg_body, 0)


_sc_forces = functools.partial(
    pl.kernel,
    out_type=jax.ShapeDtypeStruct((_NW, _B, 3 * _ROWS, _L), jnp.float32),
    mesh=plsc.VectorSubcoreMesh(core_axis_name="c", subcore_axis_name="s",
                                num_cores=_NC, num_subcores=_NS),
    scratch_types=[
        pltpu.VMEM((_SEGPAD,), jnp.int32),
        pltpu.VMEM((3 * _ROWS, _L), jnp.float32),
        pltpu.VMEM((3 * _ROWS, _L), jnp.float32),
        pltpu.VMEM((_CH,), jnp.int32),
        pltpu.VMEM((_CH,), jnp.int32),
        pltpu.VMEM((_CH,), jnp.float32),
        pltpu.VMEM((_CH,), jnp.float32),
        pltpu.VMEM((_CH,), jnp.float32),
        pltpu.VMEM((_CH,), jnp.int32),
        pltpu.VMEM((_CH,), jnp.int32),
        pltpu.VMEM((_CH,), jnp.float32),
        pltpu.VMEM((_CH,), jnp.float32),
        pltpu.VMEM((_CH,), jnp.float32),
        pltpu.SemaphoreType.DMA,
        pltpu.SemaphoreType.DMA,
    ],
    compiler_params=pltpu.CompilerParams(needs_layout_passes=False),
)(_force_body)


def kernel(relative_coordinates, basis_vectors, raw_scores_X, shifts,
           edge_src, edge_dst, edge_batch):
    b, n, d = relative_coordinates.shape
    pos = jnp.einsum('bnd,bde->bne', relative_coordinates, basis_vectors)
    pos_t = jnp.swapaxes(pos, 1, 2)
    pos_t = jnp.pad(pos_t, ((0, 0), (0, 0), (0, _NPAD - n)))
    pos_pad = pos_t.reshape(b, 3 * _ROWS, _L)
    seg = jnp.searchsorted(
        edge_batch, jnp.arange(b + 1, dtype=edge_batch.dtype)).astype(jnp.int32)
    seg_pad = jnp.concatenate(
        [seg, jnp.full((_SEGPAD - (b + 1),), _E, jnp.int32)])
    zeros = jnp.zeros((3 * _ROWS, _L), jnp.float32)
    shifts_t = shifts.T
    planes = _sc_forces(pos_pad, seg_pad,
                        edge_src.astype(jnp.int32),
                        edge_dst.astype(jnp.int32),
                        shifts_t[0], shifts_t[1], shifts_t[2], zeros)
    ck0 = jnp.minimum(jnp.arange(_NW) * _CPW, _NCHUNK)
    elo = ck0 * _CH
    ehi = jnp.minimum(ck0 + _CPW, _NCHUNK) * _CH
    mask = (jnp.minimum(seg[None, 1:], ehi[:, None])
            > jnp.maximum(seg[None, :-1], elo[:, None]))
    acc = jnp.where(mask[:, :, None, None], planes, 0.0).sum(axis=0)
    forces = jnp.transpose(acc.reshape(b, 3, _NPAD)[:, :, :n], (0, 2, 1))
    rel_forces = jnp.einsum(
        'bnd,bde->bne', forces, jnp.linalg.inv(basis_vectors))
    return raw_scores_X + rel_forces
